# Initial kernel scaffold; baseline (speedup 1.0000x reference)
import jax
import jax.numpy as jnp
from jax.experimental import pallas as pl


def kernel(seq, table):
    raise NotImplementedError("write your pallas kernel here")


# speedup vs baseline: 5.7628x; 5.7628x over previous
import functools

import jax
import jax.numpy as jnp
from jax import lax
from jax.experimental import pallas as pl
from jax.experimental.pallas import tpu as pltpu
from jax.experimental.pallas import tpu_sc as plsc

EMBED = 128
CHUNK = 128


@functools.lru_cache(maxsize=None)
def _make_kernel(n_workers, n_chunks, embed):
    b_per_w = n_chunks * CHUNK
    total = n_workers * b_per_w
    mesh = plsc.VectorSubcoreMesh(core_axis_name="c", subcore_axis_name="s")
    info = plsc.get_sparse_core_info()
    num_cores = info.num_cores

    @functools.partial(
        pl.kernel,
        mesh=mesh,
        out_type=jax.ShapeDtypeStruct((total, embed), jnp.float32),
        scratch_types=[
            pltpu.VMEM((n_chunks, CHUNK), jnp.int32),
            pltpu.VMEM((CHUNK, embed), jnp.float32),
            pltpu.SemaphoreType.DMA,
        ],
    )
    def k(idx_hbm, table_hbm, out_hbm, idx_v, rows_v, gsem):
        wid = lax.axis_index("s") * num_cores + lax.axis_index("c")
        base = wid * b_per_w
        pltpu.sync_copy(idx_hbm.at[wid], idx_v)

        def body(j, carry):
            pltpu.async_copy(table_hbm.at[idx_v.at[j]], rows_v, gsem).wait()
            off = pl.multiple_of(base + j * CHUNK, 8)
            pltpu.sync_copy(rows_v, out_hbm.at[pl.ds(off, CHUNK)])
            return carry

        lax.fori_loop(0, n_chunks, body, 0)

    return k


def kernel(seq, table):
    n_tokens = seq.shape[0] * seq.shape[1]
    n_workers = 32
    n_chunks = n_tokens // (n_workers * CHUNK)
    idx = seq.reshape(n_workers, n_chunks, CHUNK).astype(jnp.int32)
    out = _make_kernel(n_workers, n_chunks, table.shape[1])(idx, table)
    return out.reshape(seq.shape[0], seq.shape[1], table.shape[1])


# speedup vs baseline: 7.8972x; 1.3704x over previous
import functools

import jax
import jax.numpy as jnp
from jax import lax
from jax.experimental import pallas as pl
from jax.experimental.pallas import tpu as pltpu
from jax.experimental.pallas import tpu_sc as plsc

EMBED = 128
CHUNK = 128


@functools.lru_cache(maxsize=None)
def _make_kernel(n_workers, n_chunks, embed):
    b_per_w = n_chunks * CHUNK
    total = n_workers * b_per_w
    mesh = plsc.VectorSubcoreMesh(core_axis_name="c", subcore_axis_name="s")
    info = plsc.get_sparse_core_info()
    num_cores = info.num_cores

    assert n_chunks % 2 == 0 and n_chunks >= 4

    @functools.partial(
        pl.kernel,
        mesh=mesh,
        out_type=jax.ShapeDtypeStruct((total, embed), jnp.float32),
        scratch_types=[
            pltpu.VMEM((n_chunks, CHUNK), jnp.int32),
            pltpu.VMEM((CHUNK, embed), jnp.float32),
            pltpu.VMEM((CHUNK, embed), jnp.float32),
            pltpu.SemaphoreType.DMA,
            pltpu.SemaphoreType.DMA,
        ],
    )
    def k(idx_hbm, table_hbm, out_hbm, idx_v, rows0, rows1, gsem0, gsem1):
        rows = (rows0, rows1)
        gsems = (gsem0, gsem1)
        wid = lax.axis_index("s") * num_cores + lax.axis_index("c")
        base = wid * b_per_w
        pltpu.sync_copy(idx_hbm.at[wid], idx_v)

        def fire(g, b):
            pltpu.async_copy(table_hbm.at[idx_v.at[g]], rows[b], gsems[b])

        def drain_and_write(g, b):
            pltpu.make_async_copy(table_hbm.at[idx_v.at[g]], rows[b], gsems[b]).wait()
            off = pl.multiple_of(base + g * CHUNK, 8)
            pltpu.sync_copy(rows[b], out_hbm.at[pl.ds(off, CHUNK)])

        fire(0, 0)
        fire(1, 1)

        def body(i, carry):
            g = 2 * i
            for b in range(2):
                drain_and_write(g + b, b)
                fire(g + b + 2, b)
            return carry

        lax.fori_loop(0, (n_chunks - 2) // 2, body, 0)
        for b in range(2):
            drain_and_write(n_chunks - 2 + b, b)

    return k


def kernel(seq, table):
    n_tokens = seq.shape[0] * seq.shape[1]
    n_workers = 32
    n_chunks = n_tokens // (n_workers * CHUNK)
    idx = seq.reshape(n_workers, n_chunks, CHUNK).astype(jnp.int32)
    out = _make_kernel(n_workers, n_chunks, table.shape[1])(idx, table)
    return out.reshape(seq.shape[0], seq.shape[1], table.shape[1])
